# baseline (speedup 1.0000x reference)
import numpy as np
import jax
import jax.numpy as jnp
from jax import lax
from jax.experimental import pallas as pl
from jax.experimental.pallas import tpu as pltpu
from jax.experimental.pallas import tpu_sc as plsc

_NX = _NY = _NZ = 32
_D = 128
_B = 16
_NC, _NS, _L = 2, 16, 16
_NW = _NC * _NS
_QROWS = _B * _NX * _NY // 4
_QPW = _QROWS // _NW
_SROWS = 20


def _voxel_body(sa_hbm, tab_hbm, out_hbm, sa_v, idx_v, rows_v, gsem, osem):
    cid = lax.axis_index("c")
    sid = lax.axis_index("s")
    wid = sid * _NC + cid
    base = wid * _QPW

    pltpu.sync_copy(sa_hbm.at[wid], sa_v)
    lv = sa_v[0, :]
    wv = sa_v[1, :]
    hv = sa_v[2, :]

    one = jnp.ones((_L,), jnp.int32)
    zer = jnp.zeros((_L,), jnp.int32)

    rhsc_l, rhsb_l = 123.0 * lv - 512.0, 133.0 * lv - 512.0
    rhsc_w, rhsb_w = 123.0 * wv - 512.0, 133.0 * wv - 512.0
    rhsc_h, rhsb_h = 123.0 * hv - 512.0, 133.0 * hv - 512.0

    prod_z = 5.0 * hv
    iz_c = jnp.where(prod_z < rhsc_h, one, zer)
    iz_b = jnp.where(prod_z < rhsb_h + sa_v[11, :], one, zer)

    a = []
    c = []
    for t in range(4):
        prod = sa_v[5 + t, :] * wv
        ceny = prod < rhsc_w
        if t == 0:
            inby = prod < rhsb_w + sa_v[9, :]
        elif t == 3:
            inby = prod < rhsb_w + sa_v[10, :]
        else:
            inby = prod < rhsb_w
        a.append(jnp.where(inby, iz_b, zer))
        c.append(jnp.where(ceny, iz_c, zer))

    def _x3(v):
        return v + v + v

    aq = _x3(_x3(_x3(a[0]) + a[1]) + a[2]) + a[3]
    cq = _x3(_x3(_x3(c[0]) + c[1]) + c[2]) + c[3]

    for m in range(8):
        prod = sa_v[12 + m, :] * lv
        cenx = prod < rhsc_l
        if m == 0:
            rb = rhsb_l + sa_v[3, :]
        elif m == 7:
            rb = rhsb_l + sa_v[4, :]
        else:
            rb = rhsb_l
        inbx = prod < rb
        comb = jnp.where(inbx, aq, zer) + jnp.where(cenx, cq, zer)
        idx_v[m // 4, pl.ds((m % 4) * _L, _L)] = comb

    gathers = [
        pltpu.async_copy(tab_hbm.at[idx_v.at[k]],
                         rows_v.at[pl.ds(k * 64, 64)], gsem)
        for k in range(2)
    ]
    outs = []
    for k in range(2):
        gathers[k].wait()
        outs.append(
            pltpu.async_copy(rows_v.at[pl.ds(k * 64, 64)],
                             out_hbm.at[pl.ds(base + k * 64, 64)], osem))
    for o in outs:
        o.wait()


_vox = pl.kernel(
    _voxel_body,
    out_type=jax.ShapeDtypeStruct((_QROWS, 4 * _D), jnp.float32),
    mesh=plsc.VectorSubcoreMesh(core_axis_name="c", subcore_axis_name="s"),
    scratch_types=[
        pltpu.VMEM((_SROWS, _L), jnp.float32),
        pltpu.VMEM((2, 64), jnp.int32),
        pltpu.VMEM((_QPW, 4 * _D), jnp.float32),
        pltpu.SemaphoreType.DMA,
        pltpu.SemaphoreType.DMA,
    ],
)


def kernel(search_area, embed_weight):
    sa = search_area.astype(jnp.float32)

    xx = (sa - 4.0) / 1.25
    start = -sa / 2 + sa / 64
    stop = sa / 2 - sa / 64
    cgrid = jnp.linspace(start, stop, _NX, axis=-1)
    thr_b = xx / 2 + 0.5 * (sa / 32)
    bump15 = (jnp.abs(cgrid[..., 15]) < thr_b).astype(jnp.float32)
    bump16 = (jnp.abs(cgrid[..., 16]) < thr_b).astype(jnp.float32)

    widx = jnp.arange(_NW)
    samp = widx // 2
    ihalf = (widx % 2).astype(jnp.float32)
    lane = jnp.arange(_L)
    ones_l = jnp.ones((1, 1, _L), jnp.float32)
    low = (lane < 8).astype(jnp.float32)
    lwh = sa[samp][:, :, None] * ones_l
    xb0 = (ihalf * bump16[samp, 0])[:, None, None] * low[None, None, :]
    xb7 = (((1.0 - ihalf) * bump15[samp, 0])[:, None, None]
           * (1.0 - low)[None, None, :])
    ycoef = jnp.stack(
        [5.0 * jnp.abs(2.0 * (4.0 * (lane % 8) + t) - 31.0)
         for t in range(4)]).astype(jnp.float32)
    ycoef_b = jnp.broadcast_to(ycoef[None], (_NW, 4, _L))
    hotq4 = ((lane % 8) == 4).astype(jnp.float32)
    hotq3 = ((lane % 8) == 3).astype(jnp.float32)
    yb0 = bump16[samp, 1][:, None, None] * hotq4[None, None, :]
    yb3 = bump15[samp, 1][:, None, None] * hotq3[None, None, :]
    zb = bump16[samp, 2][:, None, None] * ones_l
    i0 = ihalf * 16.0
    xcoef = 5.0 * jnp.abs(
        2.0 * (i0[:, None, None] + 2.0 * jnp.arange(8.0)[None, :, None]
               + (lane // 8)[None, None, :]) - 31.0)
    blk = jnp.concatenate(
        [lwh, xb0, xb7, ycoef_b, yb0, yb3, zb, xcoef], axis=1)

    q = jnp.arange(81)
    digits = jnp.stack([q // 27 % 3, q // 9 % 3, q // 3 % 3, q % 3], axis=1)
    tab4 = embed_weight[digits].reshape(81, 4 * _D)

    out = _vox(blk, tab4)
    return out.reshape(_B, _NX, _NY, _D)


# speedup vs baseline: 3.1700x; 3.1700x over previous
import numpy as np
import jax
import jax.numpy as jnp
from jax import lax
from jax.experimental import pallas as pl
from jax.experimental.pallas import tpu as pltpu
from jax.experimental.pallas import tpu_sc as plsc

_NX = _NY = _NZ = 32
_D = 128
_B = 16
_NC, _NS, _L = 2, 16, 16
_NW = _NC * _NS
_QROWS = _B * _NX * _NY // 4
_QPW = _QROWS // _NW
_SROWS = 20


def _voxel_body(sa_hbm, tab_hbm, out_hbm, sa_v, idx_v, rows_v, gsem, osem):
    cid = lax.axis_index("c")
    sid = lax.axis_index("s")
    wid = sid * _NC + cid
    base = wid * _QPW

    pltpu.sync_copy(sa_hbm.at[wid], sa_v)
    lv = sa_v[0, :]
    wv = sa_v[1, :]
    hv = sa_v[2, :]

    one = jnp.ones((_L,), jnp.int32)
    zer = jnp.zeros((_L,), jnp.int32)

    rhsc_l, rhsb_l = 123.0 * lv - 512.0, 133.0 * lv - 512.0
    rhsc_w, rhsb_w = 123.0 * wv - 512.0, 133.0 * wv - 512.0
    rhsc_h, rhsb_h = 123.0 * hv - 512.0, 133.0 * hv - 512.0

    prod_z = 5.0 * hv
    iz_c = jnp.where(prod_z < rhsc_h, one, zer)
    iz_b = jnp.where(prod_z < rhsb_h + sa_v[11, :], one, zer)

    a = []
    c = []
    for t in range(4):
        prod = sa_v[5 + t, :] * wv
        ceny = prod < rhsc_w
        if t == 0:
            inby = prod < rhsb_w + sa_v[9, :]
        elif t == 3:
            inby = prod < rhsb_w + sa_v[10, :]
        else:
            inby = prod < rhsb_w
        a.append(jnp.where(inby, iz_b, zer))
        c.append(jnp.where(ceny, iz_c, zer))

    def _x3(v):
        return v + v + v

    aq = _x3(_x3(_x3(a[0]) + a[1]) + a[2]) + a[3]
    cq = _x3(_x3(_x3(c[0]) + c[1]) + c[2]) + c[3]

    for m in range(8):
        prod = sa_v[12 + m, :] * lv
        cenx = prod < rhsc_l
        if m == 0:
            rb = rhsb_l + sa_v[3, :]
        elif m == 7:
            rb = rhsb_l + sa_v[4, :]
        else:
            rb = rhsb_l
        inbx = prod < rb
        comb = jnp.where(inbx, aq, zer) + jnp.where(cenx, cq, zer)
        idx_v[m // 2, pl.ds((m % 2) * _L, _L)] = comb

    gathers = [
        pltpu.async_copy(tab_hbm.at[wid].at[idx_v.at[k]],
                         rows_v.at[pl.ds(k * 32, 32)], gsem)
        for k in range(4)
    ]
    outs = []
    for k in range(4):
        gathers[k].wait()
        outs.append(
            pltpu.async_copy(rows_v.at[pl.ds(k * 32, 32)],
                             out_hbm.at[pl.ds(base + k * 32, 32)], osem))
    for o in outs:
        o.wait()


_vox = pl.kernel(
    _voxel_body,
    out_type=jax.ShapeDtypeStruct((_QROWS, 4 * _D), jnp.float32),
    mesh=plsc.VectorSubcoreMesh(core_axis_name="c", subcore_axis_name="s"),
    scratch_types=[
        pltpu.VMEM((_SROWS, _L), jnp.float32),
        pltpu.VMEM((4, 32), jnp.int32),
        pltpu.VMEM((_QPW, 4 * _D), jnp.float32),
        pltpu.SemaphoreType.DMA,
        pltpu.SemaphoreType.DMA,
    ],
)


def kernel(search_area, embed_weight):
    sa = search_area.astype(jnp.float32)

    xx = (sa - 4.0) / 1.25
    start = -sa / 2 + sa / 64
    stop = sa / 2 - sa / 64
    cgrid = jnp.linspace(start, stop, _NX, axis=-1)
    thr_b = xx / 2 + 0.5 * (sa / 32)
    bump15 = (jnp.abs(cgrid[..., 15]) < thr_b).astype(jnp.float32)
    bump16 = (jnp.abs(cgrid[..., 16]) < thr_b).astype(jnp.float32)

    widx = jnp.arange(_NW)
    samp = widx // 2
    ihalf = (widx % 2).astype(jnp.float32)
    lane = jnp.arange(_L)
    ones_l = jnp.ones((1, 1, _L), jnp.float32)
    low = (lane < 8).astype(jnp.float32)
    lwh = sa[samp][:, :, None] * ones_l
    xb0 = (ihalf * bump16[samp, 0])[:, None, None] * low[None, None, :]
    xb7 = (((1.0 - ihalf) * bump15[samp, 0])[:, None, None]
           * (1.0 - low)[None, None, :])
    ycoef = jnp.stack(
        [5.0 * jnp.abs(2.0 * (4.0 * (lane % 8) + t) - 31.0)
         for t in range(4)]).astype(jnp.float32)
    ycoef_b = jnp.broadcast_to(ycoef[None], (_NW, 4, _L))
    hotq4 = ((lane % 8) == 4).astype(jnp.float32)
    hotq3 = ((lane % 8) == 3).astype(jnp.float32)
    yb0 = bump16[samp, 1][:, None, None] * hotq4[None, None, :]
    yb3 = bump15[samp, 1][:, None, None] * hotq3[None, None, :]
    zb = bump16[samp, 2][:, None, None] * ones_l
    i0 = ihalf * 16.0
    xcoef = 5.0 * jnp.abs(
        2.0 * (i0[:, None, None] + 2.0 * jnp.arange(8.0)[None, :, None]
               + (lane // 8)[None, None, :]) - 31.0)
    blk = jnp.concatenate(
        [lwh, xb0, xb7, ycoef_b, yb0, yb3, zb, xcoef], axis=1)

    q = jnp.arange(81)
    digits = jnp.stack([q // 27 % 3, q // 9 % 3, q // 3 % 3, q % 3], axis=1)
    tab4 = embed_weight[digits].reshape(81, 4 * _D)
    tab4_rep = jnp.broadcast_to(tab4[None], (_NW, 81, 4 * _D))

    out = _vox(blk, tab4_rep)
    return out.reshape(_B, _NX, _NY, _D)
